# baseline (speedup 1.0000x reference)
import jax
import jax.numpy as jnp
from jax import lax
from jax.experimental import pallas as pl
from jax.experimental.pallas import tpu as pltpu
from jax.experimental.pallas import tpu_sc as plsc

CHAR_VOCAB = 1000
WORD_VOCAB = 1000000
OUT_DIM = 64
PAD = 128
B, L = 4096, 200
N = B * L

_NC, _NS = 2, 16
NW = _NC * _NS
_NBUF = 2
_LSPLIT = 120


def _make_sc_body(rows_per_w):
    per_w = rows_per_w * L
    n_groups = rows_per_w // _NBUF

    def _sc_body(word_hbm, char_hbm, widx_hbm, cidx_hbm, wem_hbm, cem_hbm,
                 widx_v, cidx_v, wrows_v, crows_v, char_sp,
                 semw0, semw1, semc0, semc1,
                 semow0, semow1, semoc0, semoc1):
        semw = [semw0, semw1]
        semc = [semc0, semc1]
        semow = [semow0, semow1]
        semoc = [semoc0, semoc1]
        wid = lax.axis_index("s") * _NC + lax.axis_index("c")
        t0 = wid * per_w

        @pl.when(lax.axis_index("s") == 0)
        def _():
            pltpu.sync_copy(char_hbm, char_sp)
        plsc.subcore_barrier()

        pltpu.sync_copy(widx_hbm.at[pl.ds(t0, per_w)], widx_v)
        pltpu.sync_copy(cidx_hbm.at[pl.ds(t0, per_w)], cidx_v)

        lsl = [(0, _LSPLIT), (_LSPLIT, L - _LSPLIT)]

        def group(g, carry):
            i0 = g * _NBUF

            @pl.when(g > 0)
            def _():
                for p in range(_NBUF):
                    pltpu.make_async_copy(
                        wrows_v.at[p], wem_hbm.at[pl.ds(t0, L)],
                        semow[p]).wait()
                    pltpu.make_async_copy(
                        crows_v.at[p], cem_hbm.at[pl.ds(t0, L)],
                        semoc[p]).wait()

            gw = []
            gc = []
            for p in range(_NBUF):
                for (o, n) in lsl:
                    gw.append(pltpu.async_copy(
                        word_hbm.at[widx_v.at[pl.ds((i0 + p) * L + o, n)]],
                        wrows_v.at[p, pl.ds(o, n)], semw[p]))
                    gc.append(pltpu.async_copy(
                        char_sp.at[cidx_v.at[pl.ds((i0 + p) * L + o, n)]],
                        crows_v.at[p, pl.ds(o, n)], semc[p]))
            for p in range(_NBUF):
                gw[2 * p].wait()
                gw[2 * p + 1].wait()
                pltpu.async_copy(
                    wrows_v.at[p],
                    wem_hbm.at[pl.ds(t0 + (i0 + p) * L, L)], semow[p])
            for p in range(_NBUF):
                gc[2 * p].wait()
                gc[2 * p + 1].wait()
                pltpu.async_copy(
                    crows_v.at[p],
                    cem_hbm.at[pl.ds(t0 + (i0 + p) * L, L)], semoc[p])
            return carry

        lax.fori_loop(0, n_groups, group, 0)

        for p in range(_NBUF):
            pltpu.make_async_copy(
                wrows_v.at[p], wem_hbm.at[pl.ds(t0, L)], semow[p]).wait()
            pltpu.make_async_copy(
                crows_v.at[p], cem_hbm.at[pl.ds(t0, L)], semoc[p]).wait()

    return _sc_body, per_w


def _sc_gather(word_table, char_table, widx, cidx, n_rows):
    rows_per_w = n_rows // NW
    body, per_w = _make_sc_body(rows_per_w)
    n_tok = n_rows * L
    mesh = plsc.VectorSubcoreMesh(core_axis_name="c", subcore_axis_name="s")
    return pl.kernel(
        body,
        out_type=(jax.ShapeDtypeStruct((n_tok, OUT_DIM), jnp.float32),
                  jax.ShapeDtypeStruct((n_tok, OUT_DIM), jnp.float32)),
        mesh=mesh,
        scratch_types=[
            pltpu.VMEM((per_w,), jnp.int32),
            pltpu.VMEM((per_w,), jnp.int32),
            pltpu.VMEM((_NBUF, L, OUT_DIM), jnp.float32),
            pltpu.VMEM((_NBUF, L, OUT_DIM), jnp.float32),
            pltpu.VMEM_SHARED((CHAR_VOCAB, OUT_DIM), jnp.float32),
        ] + [pltpu.SemaphoreType.DMA] * (4 * _NBUF),
        compiler_params=pltpu.CompilerParams(use_tc_tiling_on_sc=False),
    )(word_table, char_table, widx, cidx)


_MIX_BLOCK = 6400


def _mix_body(w2_ref, c2_ref, w_ref, out_ref):
    out_ref[...] = (
        jnp.dot(w2_ref[...], w_ref[...], preferred_element_type=jnp.float32)
        + c2_ref[...])


def _tc_mix(wem, cem, W_dense, n_tok):
    return pl.pallas_call(
        _mix_body,
        grid=(n_tok // _MIX_BLOCK,),
        in_specs=[
            pl.BlockSpec((_MIX_BLOCK, OUT_DIM), lambda i: (i, 0)),
            pl.BlockSpec((_MIX_BLOCK, OUT_DIM), lambda i: (i, 0)),
            pl.BlockSpec((OUT_DIM, OUT_DIM), lambda i: (0, 0)),
        ],
        out_specs=pl.BlockSpec((_MIX_BLOCK, OUT_DIM), lambda i: (i, 0)),
        out_shape=jax.ShapeDtypeStruct((n_tok, OUT_DIM), jnp.float32),
    )(wem, cem, W_dense)


def kernel(char_id, word_id, char_table, word_table, W_dense):
    widx = word_id.reshape(N).astype(jnp.int32)
    cidx = char_id.reshape(N).astype(jnp.int32)
    half = B // 2
    ht = half * L
    outs = []
    for h in range(2):
        wem, cem = _sc_gather(word_table, char_table,
                              widx[h * ht:(h + 1) * ht],
                              cidx[h * ht:(h + 1) * ht], half)
        mix = _tc_mix(wem, cem, W_dense, ht)
        outs.append(mix.reshape(half, L, OUT_DIM))
    return jnp.concatenate(outs, axis=0)


# speedup vs baseline: 1.2894x; 1.2894x over previous
import jax
import jax.numpy as jnp
from jax import lax
from jax.experimental import pallas as pl
from jax.experimental.pallas import tpu as pltpu
from jax.experimental.pallas import tpu_sc as plsc

CHAR_VOCAB = 1000
WORD_VOCAB = 1000000
OUT_DIM = 64
PAD = 128
B, L = 4096, 200
N = B * L

_NC, _NS = 2, 16
NW = _NC * _NS
ROWS_PER_W = B // NW

_PROJ_BLOCK = 8000


def _proj_body(tab_ref, w_ref, out_ref):
    res = jnp.dot(tab_ref[...], w_ref[...], preferred_element_type=jnp.float32)
    out_ref[...] = jnp.concatenate(
        [res, jnp.zeros((_PROJ_BLOCK, PAD - OUT_DIM), jnp.float32)], axis=1)


def _project_table(word_table, W_dense):
    n_blocks = WORD_VOCAB // _PROJ_BLOCK
    return pl.pallas_call(
        _proj_body,
        grid=(n_blocks,),
        in_specs=[
            pl.BlockSpec((_PROJ_BLOCK, OUT_DIM), lambda i: (i, 0)),
            pl.BlockSpec((OUT_DIM, OUT_DIM), lambda i: (0, 0)),
        ],
        out_specs=pl.BlockSpec((_PROJ_BLOCK, PAD), lambda i: (i, 0)),
        out_shape=jax.ShapeDtypeStruct((WORD_VOCAB, PAD), jnp.float32),
    )(word_table, W_dense)


def _widen_body(tab_ref, out_ref):
    out_ref[...] = jnp.concatenate(
        [tab_ref[...], jnp.zeros((CHAR_VOCAB, PAD - OUT_DIM), jnp.float32)],
        axis=1)


def _widen_char(char_table):
    return pl.pallas_call(
        _widen_body,
        out_shape=jax.ShapeDtypeStruct((CHAR_VOCAB, PAD), jnp.float32),
    )(char_table)


_NBUF = 2
_LSPLIT = 120


def _make_sc_body(rows_per_w):
    per_w = rows_per_w * L
    n_groups = rows_per_w // _NBUF

    def _sc_body(proj_hbm, char_hbm, widx_hbm, cidx_hbm, out_hbm,
                 widx_v, cidx_v, rows_v, char_sp,
                 semw0, semw1, semc0, semc1, semo0, semo1):
        semw = [semw0, semw1]
        semc = [semc0, semc1]
        semo = [semo0, semo1]
        wid = lax.axis_index("s") * _NC + lax.axis_index("c")
        b0 = wid * rows_per_w
        t0 = wid * per_w

        @pl.when(lax.axis_index("s") == 0)
        def _():
            pltpu.sync_copy(char_hbm, char_sp)
        plsc.subcore_barrier()

        pltpu.sync_copy(widx_hbm.at[pl.ds(t0, per_w)], widx_v)
        pltpu.sync_copy(cidx_hbm.at[pl.ds(t0, per_w)], cidx_v)

        lsl = [(0, _LSPLIT), (_LSPLIT, L - _LSPLIT)]

        def group(g, carry):
            i0 = g * _NBUF

            @pl.when(g > 0)
            def _():
                for p in range(_NBUF):
                    pltpu.make_async_copy(
                        rows_v.at[p], out_hbm.at[pl.ds(b0 * L, L)],
                        semo[p]).wait()

            gw = []
            for p in range(_NBUF):
                for (o, n) in lsl:
                    gw.append(pltpu.async_copy(
                        proj_hbm.at[widx_v.at[pl.ds((i0 + p) * L + o, n)]],
                        rows_v.at[p, pl.ds(o, n)], semw[p]))
            ga = []
            for p in range(_NBUF):
                gw[2 * p].wait()
                gw[2 * p + 1].wait()
                for (o, n) in lsl:
                    ga.append(pltpu.async_copy(
                        char_sp.at[cidx_v.at[pl.ds((i0 + p) * L + o, n)]],
                        rows_v.at[p, pl.ds(o, n)], semc[p], add=True))
            for p in range(_NBUF):
                ga[2 * p].wait()
                ga[2 * p + 1].wait()
                pltpu.async_copy(
                    rows_v.at[p], out_hbm.at[pl.ds((b0 + i0 + p) * L, L)],
                    semo[p])
            return carry

        lax.fori_loop(0, n_groups, group, 0)

        for p in range(_NBUF):
            pltpu.make_async_copy(
                rows_v.at[p], out_hbm.at[pl.ds(b0 * L, L)], semo[p]).wait()

    return _sc_body, per_w


def _sc_mix(proj, char128, widx, cidx, n_rows):
    rows_per_w = n_rows // NW
    body, per_w = _make_sc_body(rows_per_w)
    n_tok = n_rows * L
    mesh = plsc.VectorSubcoreMesh(core_axis_name="c", subcore_axis_name="s")
    return pl.kernel(
        body,
        out_type=jax.ShapeDtypeStruct((n_tok, PAD), jnp.float32),
        mesh=mesh,
        scratch_types=[
            pltpu.VMEM((per_w,), jnp.int32),
            pltpu.VMEM((per_w,), jnp.int32),
            pltpu.VMEM((_NBUF, L, PAD), jnp.float32),
            pltpu.VMEM_SHARED((CHAR_VOCAB, PAD), jnp.float32),
        ] + [pltpu.SemaphoreType.DMA] * (3 * _NBUF),
        compiler_params=pltpu.CompilerParams(use_tc_tiling_on_sc=True),
    )(proj, char128, widx, cidx)


_RB = 16


def _repack_body(in_ref, out_ref):
    out_ref[...] = in_ref[:, :OUT_DIM].reshape(_RB, L, OUT_DIM)


def _repack_lower(mix128, n_rows):
    return pl.pallas_call(
        _repack_body,
        grid=(n_rows // _RB,),
        in_specs=[pl.BlockSpec((_RB * L, PAD), lambda i: (i, 0))],
        out_specs=pl.BlockSpec((_RB, L, OUT_DIM), lambda i: (i, 0, 0)),
        out_shape=jax.ShapeDtypeStruct((B, L, OUT_DIM), jnp.float32),
    )(mix128)


def _repack_upper_body(prev_ref, in_ref, out_ref):
    out_ref[...] = in_ref[:, :OUT_DIM].reshape(_RB, L, OUT_DIM)


def _repack_upper(prev, mix128, row0, n_rows):
    blk0 = row0 // _RB
    return pl.pallas_call(
        _repack_upper_body,
        grid=(n_rows // _RB,),
        in_specs=[
            pl.BlockSpec(memory_space=pl.ANY),
            pl.BlockSpec((_RB * L, PAD), lambda i: (i, 0)),
        ],
        out_specs=pl.BlockSpec((_RB, L, OUT_DIM),
                               lambda i, blk0=blk0: (i + blk0, 0, 0)),
        out_shape=jax.ShapeDtypeStruct((B, L, OUT_DIM), jnp.float32),
        input_output_aliases={0: 0},
    )(prev, mix128)


def kernel(char_id, word_id, char_table, word_table, W_dense):
    proj = _project_table(word_table, W_dense)
    char128 = _widen_char(char_table)
    widx = word_id.reshape(N).astype(jnp.int32)
    cidx = char_id.reshape(N).astype(jnp.int32)
    half = B // 2
    ht = half * L
    mix_a = _sc_mix(proj, char128, widx[:ht], cidx[:ht], half)
    mix_b = _sc_mix(proj, char128, widx[ht:], cidx[ht:], half)
    out = _repack_lower(mix_a, half)
    out = _repack_upper(out, mix_b, half, half)
    return out


# speedup vs baseline: 1.3213x; 1.0247x over previous
import jax
import jax.numpy as jnp
from jax import lax
from jax.experimental import pallas as pl
from jax.experimental.pallas import tpu as pltpu
from jax.experimental.pallas import tpu_sc as plsc

CHAR_VOCAB = 1000
WORD_VOCAB = 1000000
OUT_DIM = 64
PAD = 128
B, L = 4096, 200
N = B * L

_NC, _NS = 2, 16
NW = _NC * _NS
ROWS_PER_W = B // NW

_PROJ_BLOCK = 8000


def _proj_body(tab_ref, w_ref, out_ref):
    res = jnp.dot(tab_ref[...], w_ref[...], preferred_element_type=jnp.float32)
    out_ref[...] = jnp.concatenate(
        [res, jnp.zeros((_PROJ_BLOCK, PAD - OUT_DIM), jnp.float32)], axis=1)


def _project_table(word_table, W_dense):
    n_blocks = WORD_VOCAB // _PROJ_BLOCK
    return pl.pallas_call(
        _proj_body,
        grid=(n_blocks,),
        in_specs=[
            pl.BlockSpec((_PROJ_BLOCK, OUT_DIM), lambda i: (i, 0)),
            pl.BlockSpec((OUT_DIM, OUT_DIM), lambda i: (0, 0)),
        ],
        out_specs=pl.BlockSpec((_PROJ_BLOCK, PAD), lambda i: (i, 0)),
        out_shape=jax.ShapeDtypeStruct((WORD_VOCAB, PAD), jnp.float32),
    )(word_table, W_dense)


def _widen_body(tab_ref, out_ref):
    out_ref[...] = jnp.concatenate(
        [tab_ref[...], jnp.zeros((CHAR_VOCAB, PAD - OUT_DIM), jnp.float32)],
        axis=1)


def _widen_char(char_table):
    return pl.pallas_call(
        _widen_body,
        out_shape=jax.ShapeDtypeStruct((CHAR_VOCAB, PAD), jnp.float32),
    )(char_table)


_NBUF = 2
_LSPLIT = 120


def _make_sc_body(rows_per_w):
    per_w = rows_per_w * L
    n_groups = rows_per_w // _NBUF

    def _sc_body(proj_hbm, char_hbm, widx_hbm, cidx_hbm, out_hbm,
                 widx_v, cidx_v, rows_v, char_sp,
                 semw0, semw1, semc0, semc1, semo0, semo1):
        semw = [semw0, semw1]
        semc = [semc0, semc1]
        semo = [semo0, semo1]
        wid = lax.axis_index("s") * _NC + lax.axis_index("c")
        b0 = wid * rows_per_w
        t0 = wid * per_w

        @pl.when(lax.axis_index("s") == 0)
        def _():
            pltpu.sync_copy(char_hbm, char_sp)
        plsc.subcore_barrier()

        pltpu.sync_copy(widx_hbm.at[pl.ds(t0, per_w)], widx_v)
        pltpu.sync_copy(cidx_hbm.at[pl.ds(t0, per_w)], cidx_v)

        lsl = [(0, _LSPLIT), (_LSPLIT, L - _LSPLIT)]

        def group(g, carry):
            i0 = g * _NBUF

            @pl.when(g > 0)
            def _():
                for p in range(_NBUF):
                    pltpu.make_async_copy(
                        rows_v.at[p], out_hbm.at[pl.ds(b0 * L, L)],
                        semo[p]).wait()

            gw = []
            for p in range(_NBUF):
                for (o, n) in lsl:
                    gw.append(pltpu.async_copy(
                        proj_hbm.at[widx_v.at[pl.ds((i0 + p) * L + o, n)]],
                        rows_v.at[p, pl.ds(o, n)], semw[p]))
            ga = []
            for p in range(_NBUF):
                gw[2 * p].wait()
                gw[2 * p + 1].wait()
                for (o, n) in lsl:
                    ga.append(pltpu.async_copy(
                        char_sp.at[cidx_v.at[pl.ds((i0 + p) * L + o, n)]],
                        rows_v.at[p, pl.ds(o, n)], semc[p], add=True))
            for p in range(_NBUF):
                ga[2 * p].wait()
                ga[2 * p + 1].wait()
                pltpu.async_copy(
                    rows_v.at[p], out_hbm.at[pl.ds((b0 + i0 + p) * L, L)],
                    semo[p])
            return carry

        lax.fori_loop(0, n_groups, group, 0)

        for p in range(_NBUF):
            pltpu.make_async_copy(
                rows_v.at[p], out_hbm.at[pl.ds(b0 * L, L)], semo[p]).wait()

    return _sc_body, per_w


def _sc_mix(proj, char128, widx, cidx, n_rows):
    rows_per_w = n_rows // NW
    body, per_w = _make_sc_body(rows_per_w)
    n_tok = n_rows * L
    mesh = plsc.VectorSubcoreMesh(core_axis_name="c", subcore_axis_name="s")
    return pl.kernel(
        body,
        out_type=jax.ShapeDtypeStruct((n_tok, PAD), jnp.float32),
        mesh=mesh,
        scratch_types=[
            pltpu.VMEM((per_w,), jnp.int32),
            pltpu.VMEM((per_w,), jnp.int32),
            pltpu.VMEM((_NBUF, L, PAD), jnp.float32),
            pltpu.VMEM_SHARED((CHAR_VOCAB, PAD), jnp.float32),
        ] + [pltpu.SemaphoreType.DMA] * (3 * _NBUF),
        compiler_params=pltpu.CompilerParams(use_tc_tiling_on_sc=True),
    )(proj, char128, widx, cidx)


_RB = 16


def _repack_body(in_ref, out_ref):
    out_ref[...] = in_ref[:, :OUT_DIM].reshape(_RB, L, OUT_DIM)


def _repack_lower(mix128, n_rows):
    return pl.pallas_call(
        _repack_body,
        grid=(n_rows // _RB,),
        in_specs=[pl.BlockSpec((_RB * L, PAD), lambda i: (i, 0))],
        out_specs=pl.BlockSpec((_RB, L, OUT_DIM), lambda i: (i, 0, 0)),
        out_shape=jax.ShapeDtypeStruct((B, L, OUT_DIM), jnp.float32),
    )(mix128)


def _repack_upper_body(prev_ref, in_ref, out_ref):
    out_ref[...] = in_ref[:, :OUT_DIM].reshape(_RB, L, OUT_DIM)


def _repack_upper(prev, mix128, row0, n_rows):
    blk0 = row0 // _RB
    return pl.pallas_call(
        _repack_upper_body,
        grid=(n_rows // _RB,),
        in_specs=[
            pl.BlockSpec(memory_space=pl.ANY),
            pl.BlockSpec((_RB * L, PAD), lambda i: (i, 0)),
        ],
        out_specs=pl.BlockSpec((_RB, L, OUT_DIM),
                               lambda i, blk0=blk0: (i + blk0, 0, 0)),
        out_shape=jax.ShapeDtypeStruct((B, L, OUT_DIM), jnp.float32),
        input_output_aliases={0: 0},
    )(prev, mix128)


def kernel(char_id, word_id, char_table, word_table, W_dense):
    proj = _project_table(word_table, W_dense)
    char128 = _widen_char(char_table)
    widx = word_id.reshape(N).astype(jnp.int32)
    cidx = char_id.reshape(N).astype(jnp.int32)
    nsplit = 4
    part = B // nsplit
    pt = part * L
    mixes = [_sc_mix(proj, char128, widx[h * pt:(h + 1) * pt],
                     cidx[h * pt:(h + 1) * pt], part)
             for h in range(nsplit)]
    out = _repack_lower(mixes[0], part)
    for h in range(1, nsplit):
        out = _repack_upper(out, mixes[h], h * part, part)
    return out
